# baseline (speedup 1.0000x reference)
import functools

import jax
import jax.numpy as jnp
from jax.experimental import pallas as pl

_L = 4
_K = 1024
_DZ = 64
_TILE = 1024


def _vq_body(x_ref, win_ref, bin_ref, cb_ref, cbt_ref, wout_ref, bout_ref,
             out_ref):
    x = x_ref[...]
    z = jnp.dot(x, win_ref[...], preferred_element_type=jnp.float32)
    z = z + bin_ref[...]
    r = z
    t = x.shape[0]
    iota = jax.lax.broadcasted_iota(jnp.int32, (t, _K), 1)
    for l in range(_L):
        cbt = cbt_ref[l]
        cn = jnp.sum(cbt * cbt, axis=0, keepdims=True)
        scores = cn - 2.0 * jnp.dot(
            r.astype(jnp.bfloat16), cbt.astype(jnp.bfloat16),
            preferred_element_type=jnp.float32)
        m = jnp.min(scores, axis=1, keepdims=True)
        idx = jnp.min(jnp.where(scores == m, iota, _K), axis=1, keepdims=True)
        onehot = (iota == idx).astype(jnp.float32)
        q = jax.lax.dot_general(
            onehot, cb_ref[l], (((1,), (0,)), ((), ())),
            precision=jax.lax.Precision.HIGHEST,
            preferred_element_type=jnp.float32)
        r = r - q
    quant = z - r
    out = jnp.dot(quant.astype(jnp.bfloat16), wout_ref[...],
                  preferred_element_type=jnp.float32)
    out_ref[...] = out + bout_ref[...]


@functools.partial(jax.jit, static_argnames=("interpret",))
def kernel(x, proj_in_w, proj_in_b, codebooks, proj_out_w, proj_out_b,
           interpret=False):
    b, n, dims = x.shape
    tokens = b * n
    xf = x.reshape(tokens, dims).astype(jnp.bfloat16)
    cbt = jnp.swapaxes(codebooks, 1, 2)
    win = proj_in_w.astype(jnp.bfloat16)
    wout = proj_out_w.astype(jnp.bfloat16)
    bin2 = proj_in_b.reshape(1, -1)
    bout2 = proj_out_b.reshape(1, -1)
    grid = (tokens // _TILE,)
    out = pl.pallas_call(
        _vq_body,
        grid=grid,
        in_specs=[
            pl.BlockSpec((_TILE, dims), lambda i: (i, 0)),
            pl.BlockSpec((dims, _DZ), lambda i: (0, 0)),
            pl.BlockSpec((1, _DZ), lambda i: (0, 0)),
            pl.BlockSpec((_L, _K, _DZ), lambda i: (0, 0, 0)),
            pl.BlockSpec((_L, _DZ, _K), lambda i: (0, 0, 0)),
            pl.BlockSpec((_DZ, dims), lambda i: (0, 0)),
            pl.BlockSpec((1, dims), lambda i: (0, 0)),
        ],
        out_specs=pl.BlockSpec((_TILE, dims), lambda i: (i, 0)),
        out_shape=jax.ShapeDtypeStruct((tokens, dims), jnp.float32),
        interpret=interpret,
    )(xf, win, bin2, codebooks, cbt, wout, bout2)
    return out.reshape(b, n, dims)


# speedup vs baseline: 2.7070x; 2.7070x over previous
import functools

import jax
import jax.numpy as jnp
from jax.experimental import pallas as pl

_L = 4
_K = 1024
_DZ = 64
_TILE = 1024


def _vq_body(x_ref, win_ref, bin_ref, cbp_ref, cbt_ref, wout_ref, bout_ref,
             out_ref):
    x = x_ref[...]
    z = jnp.dot(x, win_ref[...], preferred_element_type=jnp.float32)
    z = z + bin_ref[...]
    r = z
    t = x.shape[0]
    iota = jax.lax.broadcasted_iota(jnp.int32, (t, _K), 1)
    for l in range(_L):
        cbt = cbt_ref[l]
        cnh = 0.5 * jnp.sum(cbt * cbt, axis=0, keepdims=True)
        s = cnh - jnp.dot(
            r.astype(jnp.bfloat16), cbt.astype(jnp.bfloat16),
            preferred_element_type=jnp.float32)
        m = jnp.min(s, axis=1, keepdims=True)
        idx = jnp.min(jnp.where(s == m, iota, _K), axis=1, keepdims=True)
        onehot = (iota == idx).astype(jnp.float32).astype(jnp.bfloat16)
        q3 = jnp.dot(onehot, cbp_ref[l],
                     preferred_element_type=jnp.float32)
        q = (q3[:, :_DZ] + q3[:, _DZ:2 * _DZ]) + q3[:, 2 * _DZ:]
        r = r - q
    quant = z - r
    out = jnp.dot(quant.astype(jnp.bfloat16), wout_ref[...],
                  preferred_element_type=jnp.float32)
    out_ref[...] = out + bout_ref[...]


@functools.partial(jax.jit, static_argnames=("interpret",))
def kernel(x, proj_in_w, proj_in_b, codebooks, proj_out_w, proj_out_b,
           interpret=False):
    b, n, dims = x.shape
    tokens = b * n
    xf = x.reshape(tokens, dims).astype(jnp.bfloat16)
    cbt = jnp.swapaxes(codebooks, 1, 2)
    hi = codebooks.astype(jnp.bfloat16)
    r1 = codebooks - hi.astype(jnp.float32)
    mid = r1.astype(jnp.bfloat16)
    lo = (r1 - mid.astype(jnp.float32)).astype(jnp.bfloat16)
    cbp = jnp.concatenate([hi, mid, lo], axis=-1)
    win = proj_in_w.astype(jnp.bfloat16)
    wout = proj_out_w.astype(jnp.bfloat16)
    bin2 = proj_in_b.reshape(1, -1)
    bout2 = proj_out_b.reshape(1, -1)
    grid = (tokens // _TILE,)
    out = pl.pallas_call(
        _vq_body,
        grid=grid,
        in_specs=[
            pl.BlockSpec((_TILE, dims), lambda i: (i, 0)),
            pl.BlockSpec((dims, _DZ), lambda i: (0, 0)),
            pl.BlockSpec((1, _DZ), lambda i: (0, 0)),
            pl.BlockSpec((_L, _K, 3 * _DZ), lambda i: (0, 0, 0)),
            pl.BlockSpec((_L, _DZ, _K), lambda i: (0, 0, 0)),
            pl.BlockSpec((_DZ, dims), lambda i: (0, 0)),
            pl.BlockSpec((1, dims), lambda i: (0, 0)),
        ],
        out_specs=pl.BlockSpec((_TILE, dims), lambda i: (i, 0)),
        out_shape=jax.ShapeDtypeStruct((tokens, dims), jnp.float32),
        interpret=interpret,
    )(xf, win, bin2, cbp, cbt, wout, bout2)
    return out.reshape(b, n, dims)
